# baseline (speedup 1.0000x reference)
import jax
import jax.numpy as jnp
from jax import lax
from jax.experimental import pallas as pl

_R = 512


def _mpnn_block(x_ref, adj_ref, wmsg_ref, wnode_ref, out_ref):
    j = pl.program_id(0)
    r = out_ref.shape[0]
    a = adj_ref[...].astype(jnp.bfloat16)
    prod = lax.dot_general(
        a, x_ref[...], (((0,), (0,)), ((), ())),
        preferred_element_type=jnp.float32,
    )
    msgsum = prod[:, 0:128]
    deg = prod[:, 128:129]
    msg = (msgsum * (1.0 / jnp.maximum(deg, 1.0))).astype(jnp.bfloat16)
    xblk = x_ref[pl.ds(j * r, r), 0:128]
    node = jnp.dot(xblk, wnode_ref[...], preferred_element_type=jnp.float32)
    msg2 = jnp.dot(msg, wmsg_ref[...], preferred_element_type=jnp.float32)
    out_ref[...] = jnp.maximum(node + msg2, 0.0)


def kernel(x, adj, W_msg, W_node):
    B, N, D = x.shape
    U = W_msg.shape[1]
    x_aug = jnp.concatenate(
        [x[0], jnp.ones((N, 1), x.dtype), jnp.zeros((N, 127), x.dtype)], axis=1
    ).astype(jnp.bfloat16)
    wmsg = W_msg.astype(jnp.bfloat16)
    wnode = W_node.astype(jnp.bfloat16)
    adj2d = adj[0]

    out = pl.pallas_call(
        _mpnn_block,
        grid=(N // _R,),
        in_specs=[
            pl.BlockSpec((N, 2 * D), lambda j: (0, 0)),
            pl.BlockSpec((N, _R), lambda j: (0, j)),
            pl.BlockSpec((D, U), lambda j: (0, 0)),
            pl.BlockSpec((D, U), lambda j: (0, 0)),
        ],
        out_specs=pl.BlockSpec((_R, U), lambda j: (j, 0)),
        out_shape=jax.ShapeDtypeStruct((N, U), jnp.float32),
    )(x_aug, adj2d, wmsg, wnode)
    return out.reshape(B, N, U)
